# baseline (speedup 1.0000x reference)
import functools

import jax
import jax.numpy as jnp
from jax.experimental import pallas as pl
from jax.experimental.pallas import tpu as pltpu

_H = 128
_NG = 16
_CUT = 10.0
_NE = 4
_NM = 1024
_NF = 128
_BM = 128
_DELTA = _CUT / (_NG - 1)
_COEFF = -0.5 / (_DELTA * _DELTA)
_SCALE = (-_COEFF) ** 0.5
_HIGH = jax.lax.Precision.HIGHEST


def _edge_kernel(tsel, x_ref, px_ref, yf_ref, qf_ref, w1_ref, b1_ref, w2_ref,
                 b2_ref, offs_ref, out_ref):
    e = _BM * _NF

    a = (jnp.dot(x_ref[...], w1_ref[0:_H, :], precision=_HIGH)
         + b1_ref[...] + tsel * w1_ref[2 * _H + _NG:2 * _H + _NG + 1, :])
    b = jnp.dot(yf_ref[...], w1_ref[_H:2 * _H, :], precision=_HIGH)

    px = px_ref[...] * _SCALE
    qt = qf_ref[...].T * _SCALE
    d2 = ((px[:, 0:1] - qt[0:1, :]) ** 2
          + (px[:, 1:2] - qt[1:2, :]) ** 2
          + (px[:, 2:3] - qt[2:3, :]) ** 2)
    d = jnp.sqrt(d2 + (1e-12 * _SCALE * _SCALE))

    u = d[:, None, :] - offs_ref[...][None, :, :]
    s = jnp.exp(-(u * u))
    g3 = jax.lax.dot_general(
        s.astype(jnp.bfloat16),
        w1_ref[2 * _H:2 * _H + _NG, :].astype(jnp.bfloat16),
        dimension_numbers=(((1,), (0,)), ((), ())),
        preferred_element_type=jnp.float32)

    pre = g3 + a[:, None, :] + b[None, :, :]
    h = jnp.maximum(pre, 0.0).reshape(e, 2 * _H)
    ot = jax.lax.dot_general(
        w2_ref[...].astype(jnp.bfloat16), h.astype(jnp.bfloat16),
        dimension_numbers=(((0,), (1,)), ((), ())),
        preferred_element_type=jnp.float32) + b2_ref[...]
    out_ref[...] = ot


def _edge_grid(tsel, x, px, yf, qf, w1, b1r, w2, b2c, offs):
    n = x.shape[0]
    e_blk = _BM * _NF
    full = lambda shape: pl.BlockSpec(shape, lambda i: (0,) * len(shape))
    dim_in = 2 * _H + _NG + 1
    return pl.pallas_call(
        functools.partial(_edge_kernel, tsel),
        grid=(n // _BM,),
        in_specs=[
            pl.BlockSpec((_BM, _H), lambda i: (i, 0)),
            pl.BlockSpec((_BM, 3), lambda i: (i, 0)),
            full((_NF, _H)),
            full((_NF, 3)),
            full((dim_in, 2 * _H)),
            full((1, 2 * _H)),
            full((2 * _H, _NE)),
            full((_NE, 1)),
            full((_NG, 1)),
        ],
        out_specs=pl.BlockSpec((_NE, e_blk), lambda i: (0, i)),
        out_shape=jax.ShapeDtypeStruct((_NE, n * _NF), jnp.float32),
        compiler_params=pltpu.CompilerParams(
            dimension_semantics=("parallel",)),
    )(x, px, yf, qf, w1, b1r, w2, b2c, offs)


@functools.partial(jax.jit, static_argnames=())
def kernel(h_mol, pos_mol, h_frag, pos_frag, batch_mol, batch_frag,
           W1, b1, W2, b2):
    offs = (jnp.arange(_NG, dtype=jnp.float32) * (_DELTA * _SCALE))[:, None]
    fft = _edge_grid(0.0, h_frag, pos_frag, h_frag, pos_frag,
                     W1, b1[None, :], W2, b2[:, None], offs)
    mft = _edge_grid(1.0, h_mol, pos_mol, h_frag, pos_frag,
                     W1, b1[None, :], W2, b2[:, None], offs)
    ff = fft.T.reshape(_NF, _NF, _NE)
    mf = mft.T.reshape(_NM, _NF, _NE)
    return ff, mf
